# baseline (speedup 1.0000x reference)
import functools

import jax
import jax.numpy as jnp
from jax import lax
from jax.experimental import pallas as pl
from jax.experimental.pallas import tpu as pltpu
from jax.experimental.pallas import tpu_sc as plsc

N = 10000
E = 320000
D = 128
DE = 4
H = 128

NC = 2
NS = 16
NW = NC * NS
CHUNK = 64
EPAD = 327680
TE0 = 13312
TE1 = 7168
SC0_EDGES = NS * TE0
NPAD = 10112
ROWS_PER_TILE = NPAD // NS

BE = 4096


def _edge_mlp_body(ea_ref, We1_ref, be1_ref, We2_ref, be2_ref, q_ref):
    a = ea_ref[...]
    h = be1_ref[...][None, :]
    for k in range(DE):
        h = h + a[:, k:k + 1] * We1_ref[k:k + 1, :]
    h = jnp.maximum(h, 0.0)
    q = (
        jnp.dot(h, We2_ref[...], preferred_element_type=jnp.float32)
        + be2_ref[...][None, :]
    )
    eid = pl.program_id(0) * BE + lax.broadcasted_iota(jnp.int32, (BE, H), 0)
    q_ref[...] = jnp.where(eid < E, q, 0.0)


def _edge_mlp(ea, We1, be1, We2, be2):
    grid = (EPAD // BE,)
    return pl.pallas_call(
        _edge_mlp_body,
        grid=grid,
        in_specs=[
            pl.BlockSpec((BE, DE), lambda i: (i, 0)),
            pl.BlockSpec((DE, H), lambda i: (0, 0)),
            pl.BlockSpec((H,), lambda i: (0,)),
            pl.BlockSpec((H, H), lambda i: (0, 0)),
            pl.BlockSpec((H,), lambda i: (0,)),
        ],
        out_specs=pl.BlockSpec((BE, H), lambda i: (i, 0)),
        out_shape=jax.ShapeDtypeStruct((EPAD, H), jnp.float32),
    )(ea, We1, be1, We2, be2)


def _node_update_body(x_ref, p_ref, W1_ref, b1_ref, W2_ref, b2_ref, o_ref):
    x = x_ref[...]
    aggr = p_ref[0, 0:N, :] + p_ref[1, 0:N, :]
    h = (
        jnp.dot(x, W1_ref[0:D, :], preferred_element_type=jnp.float32)
        + jnp.dot(aggr, W1_ref[D:2 * D, :], preferred_element_type=jnp.float32)
        + b1_ref[...][None, :]
    )
    h = jnp.maximum(h, 0.0)
    o_ref[...] = (
        x
        + jnp.dot(h, W2_ref[...], preferred_element_type=jnp.float32)
        + b2_ref[...][None, :]
    )


def _node_update(x, partials, W1, b1, W2, b2):
    return pl.pallas_call(
        _node_update_body,
        out_shape=jax.ShapeDtypeStruct((N, D), jnp.float32),
    )(x, partials, W1, b1, W2, b2)


def _sc_scatter_body(table_hbm, ii_hbm, jj_hbm, q_hbm, z_hbm, out_hbm,
                     ii_v, jj_v, q_v, rows_v, acc_sh,
                     semL0, semL1, semG0, semG1, semS0, semS1):
    cid = lax.axis_index("c")
    sid = lax.axis_index("s")
    semL = (semL0, semL1)
    semG = (semG0, semG1)
    semS = (semS0, semS1)
    te = jnp.where(cid == 0, TE0, TE1)
    tile_base = (1 - cid) * (EPAD - SC0_EDGES) + sid * te
    nsteps = te // (2 * CHUNK)

    pltpu.sync_copy(z_hbm.at[pl.ds(sid * ROWS_PER_TILE, ROWS_PER_TILE)],
                    acc_sh.at[pl.ds(sid * ROWS_PER_TILE, ROWS_PER_TILE)])
    plsc.subcore_barrier()

    def srcs(c):
        base = tile_base + c * CHUNK
        return (ii_hbm.at[pl.ds(base, CHUNK)],
                jj_hbm.at[pl.ds(base, CHUNK)],
                q_hbm.at[pl.ds(base, CHUNK)])

    def start_load(c, b):
        s_i, s_j, s_q = srcs(c)
        pltpu.async_copy(s_i, ii_v.at[b], semL[b])
        pltpu.async_copy(s_j, jj_v.at[b], semL[b])
        pltpu.async_copy(s_q, q_v.at[b], semL[b])

    def wait_load(c, b):
        s_i, s_j, s_q = srcs(c)
        pltpu.make_async_copy(s_i, ii_v.at[b], semL[b]).wait()
        pltpu.make_async_copy(s_j, jj_v.at[b], semL[b]).wait()
        pltpu.make_async_copy(s_q, q_v.at[b], semL[b]).wait()

    start_load(0, 0)
    start_load(1, 1)

    def body(it, carry):
        c = (2 * it, 2 * it + 1)
        for b in (0, 1):
            wait_load(c[b], b)
            pltpu.async_copy(table_hbm.at[jj_v.at[b]], rows_v.at[b], semG[b])
        for b in (0, 1):
            pltpu.make_async_copy(table_hbm.at[jj_v.at[b]], rows_v.at[b],
                                  semG[b]).wait()
            pltpu.async_copy(rows_v.at[b], acc_sh.at[ii_v.at[b]], semS[b],
                             add=True)
            pltpu.async_copy(q_v.at[b], acc_sh.at[ii_v.at[b]], semS[b],
                             add=True)
        for b in (0, 1):
            pltpu.make_async_copy(rows_v.at[b], acc_sh.at[ii_v.at[b]],
                                  semS[b]).wait()
            pltpu.make_async_copy(q_v.at[b], acc_sh.at[ii_v.at[b]],
                                  semS[b]).wait()

            @pl.when(it < nsteps - 1)
            def _():
                start_load(c[b] + 2, b)
        return carry

    lax.fori_loop(0, nsteps, body, 0)
    plsc.subcore_barrier()
    pltpu.sync_copy(
        acc_sh.at[pl.ds(sid * ROWS_PER_TILE, ROWS_PER_TILE)],
        out_hbm.at[pl.ds(cid * NPAD + sid * ROWS_PER_TILE, ROWS_PER_TILE)])


def _sc_scatter(table, ii, jj, q, zeros):
    mesh = plsc.VectorSubcoreMesh(core_axis_name="c", subcore_axis_name="s")
    f = pl.kernel(
        _sc_scatter_body,
        out_type=jax.ShapeDtypeStruct((NC * NPAD, D), jnp.float32),
        mesh=mesh,
        scratch_types=[
            pltpu.VMEM((2, CHUNK), jnp.int32),
            pltpu.VMEM((2, CHUNK), jnp.int32),
            pltpu.VMEM((2, CHUNK, D), jnp.float32),
            pltpu.VMEM((2, CHUNK, D), jnp.float32),
            pltpu.VMEM_SHARED((NPAD, D), jnp.float32),
            pltpu.SemaphoreType.DMA,
            pltpu.SemaphoreType.DMA,
            pltpu.SemaphoreType.DMA,
            pltpu.SemaphoreType.DMA,
            pltpu.SemaphoreType.DMA,
            pltpu.SemaphoreType.DMA,
        ],
    )
    return f(table, ii, jj, q, zeros).reshape(NC, NPAD, D)


def kernel(z, edge_index, edge_attr,
           W1_0, b1_0, W2_0, b2_0, We1_0, be1_0, We2_0, be2_0,
           W1_1, b1_1, W2_1, b2_1, We1_1, be1_1, We2_1, be2_1):
    npad = EPAD - E
    ii = jnp.concatenate([edge_index[0],
                          jnp.arange(npad, dtype=jnp.int32)])
    jj = jnp.concatenate([edge_index[1], jnp.full((npad,), N, jnp.int32)])
    ea = jnp.concatenate([edge_attr, jnp.zeros((npad, DE), jnp.float32)])
    zeros = jnp.zeros((NPAD, D), jnp.float32)
    zrow = jnp.zeros((8, D), jnp.float32)

    q0 = _edge_mlp(ea, We1_0, be1_0, We2_0, be2_0)
    q1 = _edge_mlp(ea, We1_1, be1_1, We2_1, be2_1)

    p = _sc_scatter(jnp.concatenate([z, zrow]), ii, jj, q0, zeros)
    out = _node_update(z, p, W1_0, b1_0, W2_0, b2_0)

    p = _sc_scatter(jnp.concatenate([out, zrow]), ii, jj, q1, zeros)
    out = _node_update(out, p, W1_1, b1_1, W2_1, b2_1)
    return out


# speedup vs baseline: 1.6044x; 1.6044x over previous
import functools

import jax
import jax.numpy as jnp
from jax import lax
from jax.experimental import pallas as pl
from jax.experimental.pallas import tpu as pltpu
from jax.experimental.pallas import tpu_sc as plsc

N = 10000
E = 320000
D = 128
DE = 4
H = 128

NC = 2
NS = 16
NW = NC * NS
CHUNK = 64
EPAD = 327680
TILE_EDGES = EPAD // NW
NSTEPS = TILE_EDGES // (2 * CHUNK)
NPAD = 10112
ROWS_PER_TILE = NPAD // NS

BE = 4096


def _edge_mlp_body(ea_ref, We1_ref, be1_ref, We2_ref, be2_ref, q_ref):
    a = ea_ref[...]
    h = be1_ref[...][None, :]
    for k in range(DE):
        h = h + a[:, k:k + 1] * We1_ref[k:k + 1, :]
    h = jnp.maximum(h, 0.0)
    q = (
        jnp.dot(h, We2_ref[...], preferred_element_type=jnp.float32)
        + be2_ref[...][None, :]
    )
    eid = pl.program_id(0) * BE + lax.broadcasted_iota(jnp.int32, (BE, H), 0)
    q_ref[...] = jnp.where(eid < E, q, 0.0)


def _edge_mlp(ea, We1, be1, We2, be2):
    grid = (EPAD // BE,)
    return pl.pallas_call(
        _edge_mlp_body,
        grid=grid,
        in_specs=[
            pl.BlockSpec((BE, DE), lambda i: (i, 0)),
            pl.BlockSpec((DE, H), lambda i: (0, 0)),
            pl.BlockSpec((H,), lambda i: (0,)),
            pl.BlockSpec((H, H), lambda i: (0, 0)),
            pl.BlockSpec((H,), lambda i: (0,)),
        ],
        out_specs=pl.BlockSpec((BE, H), lambda i: (i, 0)),
        out_shape=jax.ShapeDtypeStruct((EPAD, H), jnp.float32),
    )(ea, We1, be1, We2, be2)


def _node_update_body(x_ref, p_ref, W1_ref, b1_ref, W2_ref, b2_ref, o_ref):
    x = x_ref[...]
    aggr = p_ref[0, 0:N, :] + p_ref[1, 0:N, :]
    h = (
        jnp.dot(x, W1_ref[0:D, :], preferred_element_type=jnp.float32)
        + jnp.dot(aggr, W1_ref[D:2 * D, :], preferred_element_type=jnp.float32)
        + b1_ref[...][None, :]
    )
    h = jnp.maximum(h, 0.0)
    o_ref[...] = (
        x
        + jnp.dot(h, W2_ref[...], preferred_element_type=jnp.float32)
        + b2_ref[...][None, :]
    )


def _node_update(x, partials, W1, b1, W2, b2):
    return pl.pallas_call(
        _node_update_body,
        out_shape=jax.ShapeDtypeStruct((N, D), jnp.float32),
    )(x, partials, W1, b1, W2, b2)


def _sc_scatter_body(table_hbm, ii_hbm, jj_hbm, q_hbm, z_hbm, out_hbm,
                     ii_v, jj_v, q_v, rows_v, acc_sh,
                     semL0, semL1, semG0, semG1, semS0, semS1):
    cid = lax.axis_index("c")
    sid = lax.axis_index("s")
    semL = (semL0, semL1)
    semG = (semG0, semG1)
    semS = (semS0, semS1)
    tile_base = (cid * NS + sid) * TILE_EDGES

    pltpu.sync_copy(z_hbm.at[pl.ds(sid * ROWS_PER_TILE, ROWS_PER_TILE)],
                    acc_sh.at[pl.ds(sid * ROWS_PER_TILE, ROWS_PER_TILE)])
    plsc.subcore_barrier()

    def srcs(c):
        base = tile_base + c * CHUNK
        return (ii_hbm.at[pl.ds(base, CHUNK)],
                jj_hbm.at[pl.ds(base, CHUNK)],
                q_hbm.at[pl.ds(base, CHUNK)])

    def start_load(c, b):
        s_i, s_j, s_q = srcs(c)
        pltpu.async_copy(s_i, ii_v.at[b], semL[b])
        pltpu.async_copy(s_j, jj_v.at[b], semL[b])
        pltpu.async_copy(s_q, q_v.at[b], semL[b])

    def wait_load(c, b):
        s_i, s_j, s_q = srcs(c)
        pltpu.make_async_copy(s_i, ii_v.at[b], semL[b]).wait()
        pltpu.make_async_copy(s_j, jj_v.at[b], semL[b]).wait()
        pltpu.make_async_copy(s_q, q_v.at[b], semL[b]).wait()

    start_load(0, 0)
    start_load(1, 1)

    def body(it, carry):
        c = (2 * it, 2 * it + 1)
        for b in (0, 1):
            wait_load(c[b], b)
            pltpu.async_copy(table_hbm.at[jj_v.at[b]], rows_v.at[b], semG[b])
        for b in (0, 1):
            pltpu.make_async_copy(table_hbm.at[jj_v.at[b]], rows_v.at[b],
                                  semG[b]).wait()
            pltpu.async_copy(rows_v.at[b], acc_sh.at[ii_v.at[b]], semS[b],
                             add=True)
            pltpu.async_copy(q_v.at[b], acc_sh.at[ii_v.at[b]], semS[b],
                             add=True)
        for b in (0, 1):
            pltpu.make_async_copy(rows_v.at[b], acc_sh.at[ii_v.at[b]],
                                  semS[b]).wait()
            pltpu.make_async_copy(q_v.at[b], acc_sh.at[ii_v.at[b]],
                                  semS[b]).wait()

            @pl.when(it < NSTEPS - 1)
            def _():
                start_load(c[b] + 2, b)
        return carry

    lax.fori_loop(0, NSTEPS, body, 0)
    plsc.subcore_barrier()
    pltpu.sync_copy(
        acc_sh.at[pl.ds(sid * ROWS_PER_TILE, ROWS_PER_TILE)],
        out_hbm.at[pl.ds(cid * NPAD + sid * ROWS_PER_TILE, ROWS_PER_TILE)])


def _sc_scatter(table, ii, jj, q, zeros):
    mesh = plsc.VectorSubcoreMesh(core_axis_name="c", subcore_axis_name="s")
    f = pl.kernel(
        _sc_scatter_body,
        out_type=jax.ShapeDtypeStruct((NC * NPAD, D), jnp.float32),
        mesh=mesh,
        scratch_types=[
            pltpu.VMEM((2, CHUNK), jnp.int32),
            pltpu.VMEM((2, CHUNK), jnp.int32),
            pltpu.VMEM((2, CHUNK, D), jnp.float32),
            pltpu.VMEM((2, CHUNK, D), jnp.float32),
            pltpu.VMEM_SHARED((NPAD, D), jnp.float32),
            pltpu.SemaphoreType.DMA,
            pltpu.SemaphoreType.DMA,
            pltpu.SemaphoreType.DMA,
            pltpu.SemaphoreType.DMA,
            pltpu.SemaphoreType.DMA,
            pltpu.SemaphoreType.DMA,
        ],
    )
    return f(table, ii, jj, q, zeros).reshape(NC, NPAD, D)


def kernel(z, edge_index, edge_attr,
           W1_0, b1_0, W2_0, b2_0, We1_0, be1_0, We2_0, be2_0,
           W1_1, b1_1, W2_1, b2_1, We1_1, be1_1, We2_1, be2_1):
    npad = EPAD - E
    ii = jnp.concatenate([edge_index[0],
                          jnp.arange(npad, dtype=jnp.int32)])
    jj = jnp.concatenate([edge_index[1],
                          N + jnp.arange(npad, dtype=jnp.int32)])
    ea = jnp.concatenate([edge_attr, jnp.zeros((npad, DE), jnp.float32)])
    zeros = jnp.zeros((NPAD, D), jnp.float32)
    zrow = jnp.zeros((npad, D), jnp.float32)

    q0 = _edge_mlp(ea, We1_0, be1_0, We2_0, be2_0)
    q1 = _edge_mlp(ea, We1_1, be1_1, We2_1, be2_1)

    p = _sc_scatter(jnp.concatenate([z, zrow]), ii, jj, q0, zeros)
    out = _node_update(z, p, W1_0, b1_0, W2_0, b2_0)

    p = _sc_scatter(jnp.concatenate([out, zrow]), ii, jj, q1, zeros)
    out = _node_update(out, p, W1_1, b1_1, W2_1, b2_1)
    return out


# speedup vs baseline: 1.9199x; 1.1967x over previous
import functools

import jax
import jax.numpy as jnp
from jax import lax
from jax.experimental import pallas as pl
from jax.experimental.pallas import tpu as pltpu
from jax.experimental.pallas import tpu_sc as plsc

N = 10000
E = 320000
D = 128
DE = 4
H = 128

NC = 2
NS = 16
NW = NC * NS
CHUNK = 80
TILE_EDGES = E // NW
NCHUNKS = TILE_EDGES // CHUNK
NSTEPS = NCHUNKS // 2
NPAD = 10112
ROWS_PER_TILE = NPAD // NS

BE = 4000


def _edge_mlp_body(ea_ref, We1_ref, be1_ref, We2_ref, be2_ref, q_ref):
    a = ea_ref[...]
    h = be1_ref[...][None, :]
    for k in range(DE):
        h = h + a[:, k:k + 1] * We1_ref[k:k + 1, :]
    h = jnp.maximum(h, 0.0)
    q_ref[...] = (
        jnp.dot(h, We2_ref[...], preferred_element_type=jnp.float32)
        + be2_ref[...][None, :]
    )


def _edge_mlp(ea, We1, be1, We2, be2):
    grid = (E // BE,)
    return pl.pallas_call(
        _edge_mlp_body,
        grid=grid,
        in_specs=[
            pl.BlockSpec((BE, DE), lambda i: (i, 0)),
            pl.BlockSpec((DE, H), lambda i: (0, 0)),
            pl.BlockSpec((H,), lambda i: (0,)),
            pl.BlockSpec((H, H), lambda i: (0, 0)),
            pl.BlockSpec((H,), lambda i: (0,)),
        ],
        out_specs=pl.BlockSpec((BE, H), lambda i: (i, 0)),
        out_shape=jax.ShapeDtypeStruct((E, H), jnp.float32),
    )(ea, We1, be1, We2, be2)


def _node_update_body(x_ref, p_ref, W1_ref, b1_ref, W2_ref, b2_ref, o_ref):
    x = x_ref[...]
    aggr = p_ref[0, 0:N, :] + p_ref[1, 0:N, :]
    h = (
        jnp.dot(x, W1_ref[0:D, :], preferred_element_type=jnp.float32)
        + jnp.dot(aggr, W1_ref[D:2 * D, :], preferred_element_type=jnp.float32)
        + b1_ref[...][None, :]
    )
    h = jnp.maximum(h, 0.0)
    o_ref[...] = (
        x
        + jnp.dot(h, W2_ref[...], preferred_element_type=jnp.float32)
        + b2_ref[...][None, :]
    )


def _node_update(x, partials, W1, b1, W2, b2):
    return pl.pallas_call(
        _node_update_body,
        out_shape=jax.ShapeDtypeStruct((N, D), jnp.float32),
    )(x, partials, W1, b1, W2, b2)


def _sc_scatter_body(table_hbm, ii_hbm, jj_hbm, q_hbm, z_hbm, out_hbm,
                     ii_v, jj_v, q_v, rows_v, acc_sh,
                     semL0, semL1, semG0, semG1, semS0, semS1):
    cid = lax.axis_index("c")
    sid = lax.axis_index("s")
    semL = (semL0, semL1)
    semG = (semG0, semG1)
    semS = (semS0, semS1)
    tile_base = (cid * NS + sid) * TILE_EDGES

    pltpu.sync_copy(z_hbm.at[pl.ds(sid * ROWS_PER_TILE, ROWS_PER_TILE)],
                    acc_sh.at[pl.ds(sid * ROWS_PER_TILE, ROWS_PER_TILE)])
    plsc.subcore_barrier()

    def srcs(c):
        base = tile_base + c * CHUNK
        return (ii_hbm.at[pl.ds(base, CHUNK)],
                jj_hbm.at[pl.ds(base, CHUNK)],
                q_hbm.at[pl.ds(base, CHUNK)])

    def start_load(c, b):
        s_i, s_j, s_q = srcs(c)
        pltpu.async_copy(s_i, ii_v.at[b], semL[b])
        pltpu.async_copy(s_j, jj_v.at[b], semL[b])
        pltpu.async_copy(s_q, q_v.at[b], semL[b])

    def wait_load(c, b):
        s_i, s_j, s_q = srcs(c)
        pltpu.make_async_copy(s_i, ii_v.at[b], semL[b]).wait()
        pltpu.make_async_copy(s_j, jj_v.at[b], semL[b]).wait()
        pltpu.make_async_copy(s_q, q_v.at[b], semL[b]).wait()

    start_load(0, 0)
    start_load(1, 1)

    def body(it, carry):
        c = (2 * it, 2 * it + 1)
        for b in (0, 1):
            wait_load(c[b], b)
            pltpu.async_copy(table_hbm.at[jj_v.at[b]], rows_v.at[b], semG[b])
        for b in (0, 1):
            pltpu.make_async_copy(table_hbm.at[jj_v.at[b]], rows_v.at[b],
                                  semG[b]).wait()
            pltpu.async_copy(rows_v.at[b], acc_sh.at[ii_v.at[b]], semS[b],
                             add=True)
            pltpu.async_copy(q_v.at[b], acc_sh.at[ii_v.at[b]], semS[b],
                             add=True)
        for b in (0, 1):
            pltpu.make_async_copy(rows_v.at[b], acc_sh.at[ii_v.at[b]],
                                  semS[b]).wait()
            pltpu.make_async_copy(q_v.at[b], acc_sh.at[ii_v.at[b]],
                                  semS[b]).wait()

            @pl.when(c[b] + 2 < NCHUNKS)
            def _():
                start_load(c[b] + 2, b)
        return carry

    lax.fori_loop(0, NSTEPS, body, 0)
    cl = NCHUNKS - 1
    wait_load(cl, 0)
    pltpu.async_copy(table_hbm.at[jj_v.at[0]], rows_v.at[0], semG0)
    pltpu.make_async_copy(table_hbm.at[jj_v.at[0]], rows_v.at[0], semG0).wait()
    pltpu.async_copy(rows_v.at[0], acc_sh.at[ii_v.at[0]], semS0, add=True)
    pltpu.async_copy(q_v.at[0], acc_sh.at[ii_v.at[0]], semS0, add=True)
    pltpu.make_async_copy(rows_v.at[0], acc_sh.at[ii_v.at[0]], semS0).wait()
    pltpu.make_async_copy(q_v.at[0], acc_sh.at[ii_v.at[0]], semS0).wait()
    plsc.subcore_barrier()
    pltpu.sync_copy(
        acc_sh.at[pl.ds(sid * ROWS_PER_TILE, ROWS_PER_TILE)],
        out_hbm.at[pl.ds(cid * NPAD + sid * ROWS_PER_TILE, ROWS_PER_TILE)])


def _sc_scatter(table, ii, jj, q, zeros):
    mesh = plsc.VectorSubcoreMesh(core_axis_name="c", subcore_axis_name="s")
    f = pl.kernel(
        _sc_scatter_body,
        out_type=jax.ShapeDtypeStruct((NC * NPAD, D), jnp.float32),
        mesh=mesh,
        scratch_types=[
            pltpu.VMEM((2, CHUNK), jnp.int32),
            pltpu.VMEM((2, CHUNK), jnp.int32),
            pltpu.VMEM((2, CHUNK, D), jnp.float32),
            pltpu.VMEM((2, CHUNK, D), jnp.float32),
            pltpu.VMEM_SHARED((NPAD, D), jnp.float32),
            pltpu.SemaphoreType.DMA,
            pltpu.SemaphoreType.DMA,
            pltpu.SemaphoreType.DMA,
            pltpu.SemaphoreType.DMA,
            pltpu.SemaphoreType.DMA,
            pltpu.SemaphoreType.DMA,
        ],
    )
    return f(table, ii, jj, q, zeros).reshape(NC, NPAD, D)


def kernel(z, edge_index, edge_attr,
           W1_0, b1_0, W2_0, b2_0, We1_0, be1_0, We2_0, be2_0,
           W1_1, b1_1, W2_1, b2_1, We1_1, be1_1, We2_1, be2_1):
    zeros = jnp.zeros((NPAD, D), jnp.float32)
    ii = edge_index[0]
    jj = edge_index[1]

    q0 = _edge_mlp(edge_attr, We1_0, be1_0, We2_0, be2_0)
    q1 = _edge_mlp(edge_attr, We1_1, be1_1, We2_1, be2_1)

    p = _sc_scatter(z, ii, jj, q0, zeros)
    out = _node_update(z, p, W1_0, b1_0, W2_0, b2_0)

    p = _sc_scatter(out, ii, jj, q1, zeros)
    out = _node_update(out, p, W1_1, b1_1, W2_1, b2_1)
    return out


# speedup vs baseline: 2.5056x; 1.3051x over previous
import functools

import jax
import jax.numpy as jnp
from jax import lax
from jax.experimental import pallas as pl
from jax.experimental.pallas import tpu as pltpu
from jax.experimental.pallas import tpu_sc as plsc

N = 10000
E = 320000
D = 128
DE = 4
H = 128

NC = 2
NS = 16
NW = NC * NS
CHUNK = 80
TILE_EDGES = E // NW
NCHUNKS = TILE_EDGES // CHUNK
NSTEPS = NCHUNKS // 2
NPAD = 10112
ROWS_PER_TILE = NPAD // NS

BE = 12800


def _edge_mlp_body(ea_ref, We1_ref, be1_ref, We2_ref, be2_ref, q_ref):
    a = ea_ref[...]
    h = lax.dot_general(a, We1_ref[...], (((0,), (0,)), ((), ())),
                        preferred_element_type=jnp.float32)
    h = jnp.maximum(h + be1_ref[...][None, :], 0.0)
    q_ref[...] = (
        jnp.dot(h.astype(jnp.bfloat16), We2_ref[...].astype(jnp.bfloat16),
                preferred_element_type=jnp.float32)
        + be2_ref[...][None, :]
    )


def _edge_mlp(ea, We1, be1, We2, be2):
    grid = (E // BE,)
    return pl.pallas_call(
        _edge_mlp_body,
        grid=grid,
        in_specs=[
            pl.BlockSpec((DE, BE), lambda i: (0, i)),
            pl.BlockSpec((DE, H), lambda i: (0, 0)),
            pl.BlockSpec((H,), lambda i: (0,)),
            pl.BlockSpec((H, H), lambda i: (0, 0)),
            pl.BlockSpec((H,), lambda i: (0,)),
        ],
        out_specs=pl.BlockSpec((BE, H), lambda i: (i, 0)),
        out_shape=jax.ShapeDtypeStruct((E, H), jnp.float32),
    )(ea, We1, be1, We2, be2)


def _node_update_body(x_ref, p_ref, W1_ref, b1_ref, W2_ref, b2_ref, o_ref):
    x = x_ref[...]
    aggr = p_ref[0, 0:N, :] + p_ref[1, 0:N, :]
    h = (
        jnp.dot(x, W1_ref[0:D, :], preferred_element_type=jnp.float32)
        + jnp.dot(aggr, W1_ref[D:2 * D, :], preferred_element_type=jnp.float32)
        + b1_ref[...][None, :]
    )
    h = jnp.maximum(h, 0.0)
    o_ref[...] = (
        x
        + jnp.dot(h, W2_ref[...], preferred_element_type=jnp.float32)
        + b2_ref[...][None, :]
    )


def _node_update(x, partials, W1, b1, W2, b2):
    return pl.pallas_call(
        _node_update_body,
        out_shape=jax.ShapeDtypeStruct((N, D), jnp.float32),
    )(x, partials, W1, b1, W2, b2)


def _sc_scatter_body(table_hbm, ii_hbm, jj_hbm, q_hbm, z_hbm, out_hbm,
                     ii_v, jj_v, q_v, rows_v, acc_sh,
                     semL0, semL1, semG0, semG1, semS0, semS1):
    cid = lax.axis_index("c")
    sid = lax.axis_index("s")
    semL = (semL0, semL1)
    semG = (semG0, semG1)
    semS = (semS0, semS1)
    tile_base = (cid * NS + sid) * TILE_EDGES

    pltpu.sync_copy(z_hbm.at[pl.ds(sid * ROWS_PER_TILE, ROWS_PER_TILE)],
                    acc_sh.at[pl.ds(sid * ROWS_PER_TILE, ROWS_PER_TILE)])
    plsc.subcore_barrier()

    def srcs(c):
        base = tile_base + c * CHUNK
        return (ii_hbm.at[pl.ds(base, CHUNK)],
                jj_hbm.at[pl.ds(base, CHUNK)],
                q_hbm.at[pl.ds(base, CHUNK)])

    def start_load(c, b):
        s_i, s_j, s_q = srcs(c)
        pltpu.async_copy(s_i, ii_v.at[b], semL[b])
        pltpu.async_copy(s_j, jj_v.at[b], semL[b])
        pltpu.async_copy(s_q, q_v.at[b], semL[b])

    def wait_load(c, b):
        s_i, s_j, s_q = srcs(c)
        pltpu.make_async_copy(s_i, ii_v.at[b], semL[b]).wait()
        pltpu.make_async_copy(s_j, jj_v.at[b], semL[b]).wait()
        pltpu.make_async_copy(s_q, q_v.at[b], semL[b]).wait()

    start_load(0, 0)
    start_load(1, 1)

    def body(it, carry):
        c = (2 * it, 2 * it + 1)
        for b in (0, 1):
            wait_load(c[b], b)
            pltpu.async_copy(table_hbm.at[jj_v.at[b]], rows_v.at[b], semG[b])
        for b in (0, 1):
            pltpu.make_async_copy(table_hbm.at[jj_v.at[b]], rows_v.at[b],
                                  semG[b]).wait()
            pltpu.async_copy(rows_v.at[b], acc_sh.at[ii_v.at[b]], semS[b],
                             add=True)
            pltpu.async_copy(q_v.at[b], acc_sh.at[ii_v.at[b]], semS[b],
                             add=True)
        for b in (0, 1):
            pltpu.make_async_copy(rows_v.at[b], acc_sh.at[ii_v.at[b]],
                                  semS[b]).wait()
            pltpu.make_async_copy(q_v.at[b], acc_sh.at[ii_v.at[b]],
                                  semS[b]).wait()

            @pl.when(c[b] + 2 < NCHUNKS)
            def _():
                start_load(c[b] + 2, b)
        return carry

    lax.fori_loop(0, NSTEPS, body, 0)
    cl = NCHUNKS - 1
    wait_load(cl, 0)
    pltpu.async_copy(table_hbm.at[jj_v.at[0]], rows_v.at[0], semG0)
    pltpu.make_async_copy(table_hbm.at[jj_v.at[0]], rows_v.at[0], semG0).wait()
    pltpu.async_copy(rows_v.at[0], acc_sh.at[ii_v.at[0]], semS0, add=True)
    pltpu.async_copy(q_v.at[0], acc_sh.at[ii_v.at[0]], semS0, add=True)
    pltpu.make_async_copy(rows_v.at[0], acc_sh.at[ii_v.at[0]], semS0).wait()
    pltpu.make_async_copy(q_v.at[0], acc_sh.at[ii_v.at[0]], semS0).wait()
    plsc.subcore_barrier()
    pltpu.sync_copy(
        acc_sh.at[pl.ds(sid * ROWS_PER_TILE, ROWS_PER_TILE)],
        out_hbm.at[pl.ds(cid * NPAD + sid * ROWS_PER_TILE, ROWS_PER_TILE)])


def _sc_scatter(table, ii, jj, q, zeros):
    mesh = plsc.VectorSubcoreMesh(core_axis_name="c", subcore_axis_name="s")
    f = pl.kernel(
        _sc_scatter_body,
        out_type=jax.ShapeDtypeStruct((NC * NPAD, D), jnp.float32),
        mesh=mesh,
        scratch_types=[
            pltpu.VMEM((2, CHUNK), jnp.int32),
            pltpu.VMEM((2, CHUNK), jnp.int32),
            pltpu.VMEM((2, CHUNK, D), jnp.float32),
            pltpu.VMEM((2, CHUNK, D), jnp.float32),
            pltpu.VMEM_SHARED((NPAD, D), jnp.float32),
            pltpu.SemaphoreType.DMA,
            pltpu.SemaphoreType.DMA,
            pltpu.SemaphoreType.DMA,
            pltpu.SemaphoreType.DMA,
            pltpu.SemaphoreType.DMA,
            pltpu.SemaphoreType.DMA,
        ],
    )
    return f(table, ii, jj, q, zeros).reshape(NC, NPAD, D)


def kernel(z, edge_index, edge_attr,
           W1_0, b1_0, W2_0, b2_0, We1_0, be1_0, We2_0, be2_0,
           W1_1, b1_1, W2_1, b2_1, We1_1, be1_1, We2_1, be2_1):
    zeros = jnp.zeros((NPAD, D), jnp.float32)
    ii = edge_index[0]
    jj = edge_index[1]

    ea_t = edge_attr.T
    q0 = _edge_mlp(ea_t, We1_0, be1_0, We2_0, be2_0)
    q1 = _edge_mlp(ea_t, We1_1, be1_1, We2_1, be2_1)

    p = _sc_scatter(z, ii, jj, q0, zeros)
    out = _node_update(z, p, W1_0, b1_0, W2_0, b2_0)

    p = _sc_scatter(out, ii, jj, q1, zeros)
    out = _node_update(out, p, W1_1, b1_1, W2_1, b2_1)
    return out


# speedup vs baseline: 2.7577x; 1.1006x over previous
import functools

import jax
import jax.numpy as jnp
from jax import lax
from jax.experimental import pallas as pl
from jax.experimental.pallas import tpu as pltpu
from jax.experimental.pallas import tpu_sc as plsc

N = 10000
E = 320000
D = 128
DE = 4
H = 128

NC = 2
NS = 16
NW = NC * NS
CHUNK = 80
TILE_EDGES = E // NW
NCHUNKS = TILE_EDGES // CHUNK
NSTEPS = NCHUNKS // 2
NPAD = 10112
ROWS_PER_TILE = NPAD // NS

BE = 12800


def _edge_mlp_body(ea_ref, We1_ref, be1_ref, We2_ref, be2_ref, q_ref):
    a = ea_ref[...]
    h = lax.dot_general(a, We1_ref[...], (((0,), (0,)), ((), ())),
                        preferred_element_type=jnp.float32)
    h = jnp.maximum(h + be1_ref[...][None, :], 0.0)
    q_ref[...] = (
        jnp.dot(h.astype(jnp.bfloat16), We2_ref[...].astype(jnp.bfloat16),
                preferred_element_type=jnp.float32)
        + be2_ref[...][None, :]
    )


def _edge_mlp(ea, We1, be1, We2, be2):
    grid = (E // BE,)
    return pl.pallas_call(
        _edge_mlp_body,
        grid=grid,
        in_specs=[
            pl.BlockSpec((DE, BE), lambda i: (0, i)),
            pl.BlockSpec((DE, H), lambda i: (0, 0)),
            pl.BlockSpec((H,), lambda i: (0,)),
            pl.BlockSpec((H, H), lambda i: (0, 0)),
            pl.BlockSpec((H,), lambda i: (0,)),
        ],
        out_specs=pl.BlockSpec((BE, H), lambda i: (i, 0)),
        out_shape=jax.ShapeDtypeStruct((E, H), jnp.float32),
    )(ea, We1, be1, We2, be2)


def _node_update_body(x_ref, p_ref, W1_ref, b1_ref, W2_ref, b2_ref, o_ref):
    x = x_ref[...]
    aggr = p_ref[0, 0:N, :] + p_ref[1, 0:N, :]
    h = (
        jnp.dot(x, W1_ref[0:D, :], preferred_element_type=jnp.float32)
        + jnp.dot(aggr, W1_ref[D:2 * D, :], preferred_element_type=jnp.float32)
        + b1_ref[...][None, :]
    )
    h = jnp.maximum(h, 0.0)
    o_ref[...] = (
        x
        + jnp.dot(h, W2_ref[...], preferred_element_type=jnp.float32)
        + b2_ref[...][None, :]
    )


def _node_update(x, partials, W1, b1, W2, b2):
    return pl.pallas_call(
        _node_update_body,
        out_shape=jax.ShapeDtypeStruct((N, D), jnp.float32),
    )(x, partials, W1, b1, W2, b2)


def _sc_scatter_body(table_hbm, ii_hbm, jj_hbm, q_hbm, z_hbm, out_hbm,
                     ii_v, jj_v, q_v, rows_v, acc_sh,
                     semL0, semL1, semG0, semG1, semS0, semS1):
    cid = lax.axis_index("c")
    sid = lax.axis_index("s")
    semL = (semL0, semL1)
    semG = (semG0, semG1)
    semS = (semS0, semS1)
    tile_base = (cid * NS + sid) * TILE_EDGES

    pltpu.sync_copy(z_hbm.at[pl.ds(sid * ROWS_PER_TILE, ROWS_PER_TILE)],
                    acc_sh.at[pl.ds(sid * ROWS_PER_TILE, ROWS_PER_TILE)])
    plsc.subcore_barrier()

    def srcs(c):
        base = tile_base + c * CHUNK
        return (ii_hbm.at[pl.ds(base, CHUNK)],
                jj_hbm.at[pl.ds(base, CHUNK)],
                q_hbm.at[pl.ds(base, CHUNK)])

    def start_load(c, b):
        s_i, s_j, s_q = srcs(c)
        pltpu.async_copy(s_i, ii_v.at[b], semL[b])
        pltpu.async_copy(s_j, jj_v.at[b], semL[b])
        pltpu.async_copy(s_q, q_v.at[b], semL[b])

    def wait_load(c, b):
        s_i, s_j, s_q = srcs(c)
        pltpu.make_async_copy(s_i, ii_v.at[b], semL[b]).wait()
        pltpu.make_async_copy(s_j, jj_v.at[b], semL[b]).wait()
        pltpu.make_async_copy(s_q, q_v.at[b], semL[b]).wait()

    start_load(0, 0)
    start_load(1, 1)

    def body(it, carry):
        c = (2 * it, 2 * it + 1)
        for b in (0, 1):
            wait_load(c[b], b)
            pltpu.async_copy(table_hbm.at[jj_v.at[b]], rows_v.at[b], semG[b])
        for b in (0, 1):
            pltpu.make_async_copy(table_hbm.at[jj_v.at[b]], rows_v.at[b],
                                  semG[b]).wait()

            def add_body(e, carry2, b=b):
                for k in range(D // 16):
                    sl = pl.ds(k * 16, 16)
                    rows_v[b, e, sl] = rows_v[b, e, sl] + q_v[b, e, sl]
                return carry2

            lax.fori_loop(0, CHUNK, add_body, 0)
            pltpu.async_copy(rows_v.at[b], acc_sh.at[ii_v.at[b]], semS[b],
                             add=True)
        for b in (0, 1):
            pltpu.make_async_copy(rows_v.at[b], acc_sh.at[ii_v.at[b]],
                                  semS[b]).wait()

            @pl.when(c[b] + 2 < NCHUNKS)
            def _():
                start_load(c[b] + 2, b)
        return carry

    lax.fori_loop(0, NSTEPS, body, 0)
    cl = NCHUNKS - 1
    wait_load(cl, 0)
    pltpu.async_copy(table_hbm.at[jj_v.at[0]], rows_v.at[0], semG0)
    pltpu.make_async_copy(table_hbm.at[jj_v.at[0]], rows_v.at[0], semG0).wait()

    def add_body_l(e, carry2):
        for k in range(D // 16):
            sl = pl.ds(k * 16, 16)
            rows_v[0, e, sl] = rows_v[0, e, sl] + q_v[0, e, sl]
        return carry2

    lax.fori_loop(0, CHUNK, add_body_l, 0)
    pltpu.async_copy(rows_v.at[0], acc_sh.at[ii_v.at[0]], semS0, add=True)
    pltpu.make_async_copy(rows_v.at[0], acc_sh.at[ii_v.at[0]], semS0).wait()
    plsc.subcore_barrier()
    pltpu.sync_copy(
        acc_sh.at[pl.ds(sid * ROWS_PER_TILE, ROWS_PER_TILE)],
        out_hbm.at[pl.ds(cid * NPAD + sid * ROWS_PER_TILE, ROWS_PER_TILE)])


def _sc_scatter(table, ii, jj, q, zeros):
    mesh = plsc.VectorSubcoreMesh(core_axis_name="c", subcore_axis_name="s")
    f = pl.kernel(
        _sc_scatter_body,
        out_type=jax.ShapeDtypeStruct((NC * NPAD, D), jnp.float32),
        mesh=mesh,
        scratch_types=[
            pltpu.VMEM((2, CHUNK), jnp.int32),
            pltpu.VMEM((2, CHUNK), jnp.int32),
            pltpu.VMEM((2, CHUNK, D), jnp.float32),
            pltpu.VMEM((2, CHUNK, D), jnp.float32),
            pltpu.VMEM_SHARED((NPAD, D), jnp.float32),
            pltpu.SemaphoreType.DMA,
            pltpu.SemaphoreType.DMA,
            pltpu.SemaphoreType.DMA,
            pltpu.SemaphoreType.DMA,
            pltpu.SemaphoreType.DMA,
            pltpu.SemaphoreType.DMA,
        ],
    )
    return f(table, ii, jj, q, zeros).reshape(NC, NPAD, D)


def kernel(z, edge_index, edge_attr,
           W1_0, b1_0, W2_0, b2_0, We1_0, be1_0, We2_0, be2_0,
           W1_1, b1_1, W2_1, b2_1, We1_1, be1_1, We2_1, be2_1):
    zeros = jnp.zeros((NPAD, D), jnp.float32)
    ii = edge_index[0]
    jj = edge_index[1]

    ea_t = edge_attr.T
    q0 = _edge_mlp(ea_t, We1_0, be1_0, We2_0, be2_0)
    q1 = _edge_mlp(ea_t, We1_1, be1_1, We2_1, be2_1)

    p = _sc_scatter(z, ii, jj, q0, zeros)
    out = _node_update(z, p, W1_0, b1_0, W2_0, b2_0)

    p = _sc_scatter(out, ii, jj, q1, zeros)
    out = _node_update(out, p, W1_1, b1_1, W2_1, b2_1)
    return out


# speedup vs baseline: 2.9877x; 1.0834x over previous
import functools

import jax
import jax.numpy as jnp
from jax import lax
from jax.experimental import pallas as pl
from jax.experimental.pallas import tpu as pltpu
from jax.experimental.pallas import tpu_sc as plsc

N = 10000
E = 320000
D = 128
DE = 4
H = 128

NC = 2
NS = 16
NW = NC * NS
CHUNK = 80
TILE_EDGES = E // NW
NCHUNKS = TILE_EDGES // CHUNK
NSTEPS = NCHUNKS // 2
NPAD = 10112
ROWS_PER_TILE = NPAD // NS

BE = 12800


def _edge_mlp_body(ea_ref, We1_ref, be1_ref, We2_ref, be2_ref, q_ref):
    a = ea_ref[...]
    h = lax.dot_general(a, We1_ref[...], (((0,), (0,)), ((), ())),
                        preferred_element_type=jnp.float32)
    h = jnp.maximum(h + be1_ref[...][None, :], 0.0)
    q_ref[...] = (
        jnp.dot(h.astype(jnp.bfloat16), We2_ref[...].astype(jnp.bfloat16),
                preferred_element_type=jnp.float32)
        + be2_ref[...][None, :]
    )


def _edge_mlp(ea, We1, be1, We2, be2):
    grid = (E // BE,)
    return pl.pallas_call(
        _edge_mlp_body,
        grid=grid,
        in_specs=[
            pl.BlockSpec((DE, BE), lambda i: (0, i)),
            pl.BlockSpec((DE, H), lambda i: (0, 0)),
            pl.BlockSpec((H,), lambda i: (0,)),
            pl.BlockSpec((H, H), lambda i: (0, 0)),
            pl.BlockSpec((H,), lambda i: (0,)),
        ],
        out_specs=pl.BlockSpec((BE, H), lambda i: (i, 0)),
        out_shape=jax.ShapeDtypeStruct((E, H), jnp.float32),
    )(ea, We1, be1, We2, be2)


def _node_update_body(x_ref, p_ref, W1_ref, b1_ref, W2_ref, b2_ref, o_ref):
    x = x_ref[...]
    aggr = p_ref[0, 0:N, :] + p_ref[1, 0:N, :]
    h = (
        jnp.dot(x, W1_ref[0:D, :], preferred_element_type=jnp.float32)
        + jnp.dot(aggr, W1_ref[D:2 * D, :], preferred_element_type=jnp.float32)
        + b1_ref[...][None, :]
    )
    h = jnp.maximum(h, 0.0)
    o_ref[...] = (
        x
        + jnp.dot(h, W2_ref[...], preferred_element_type=jnp.float32)
        + b2_ref[...][None, :]
    )


def _node_update(x, partials, W1, b1, W2, b2):
    return pl.pallas_call(
        _node_update_body,
        out_shape=jax.ShapeDtypeStruct((N, D), jnp.float32),
    )(x, partials, W1, b1, W2, b2)


def _sc_scatter_body(table_hbm, ii_hbm, jj_hbm, q_hbm, z_hbm, out_hbm,
                     ii_v, jj_v, q_v, rows_v, acc_sh,
                     semI0, semI1, semJ0, semJ1, semQ0, semQ1,
                     semG0, semG1, semS0, semS1):
    cid = lax.axis_index("c")
    sid = lax.axis_index("s")
    semI = (semI0, semI1)
    semJ = (semJ0, semJ1)
    semQ = (semQ0, semQ1)
    semG = (semG0, semG1)
    semS = (semS0, semS1)
    tile_base = (cid * NS + sid) * TILE_EDGES

    pltpu.sync_copy(z_hbm.at[pl.ds(sid * ROWS_PER_TILE, ROWS_PER_TILE)],
                    acc_sh.at[pl.ds(sid * ROWS_PER_TILE, ROWS_PER_TILE)])
    plsc.subcore_barrier()

    def srcs(c):
        base = tile_base + c * CHUNK
        return (ii_hbm.at[pl.ds(base, CHUNK)],
                jj_hbm.at[pl.ds(base, CHUNK)],
                q_hbm.at[pl.ds(base, CHUNK)])

    def start_jq(c, b):
        s_i, s_j, s_q = srcs(c)
        pltpu.async_copy(s_j, jj_v.at[b], semJ[b])
        pltpu.async_copy(s_q, q_v.at[b], semQ[b])

    def start_i(c, b):
        s_i, s_j, s_q = srcs(c)
        pltpu.async_copy(s_i, ii_v.at[b], semI[b])

    def wait(which, c, b):
        s_i, s_j, s_q = srcs(c)
        src, dst, sem = {
            "i": (s_i, ii_v, semI), "j": (s_j, jj_v, semJ),
            "q": (s_q, q_v, semQ),
        }[which]
        pltpu.make_async_copy(src, dst.at[b], sem[b]).wait()

    def gather_start(b):
        pltpu.async_copy(table_hbm.at[jj_v.at[b]], rows_v.at[b], semG[b])

    def gather_wait(b):
        pltpu.make_async_copy(table_hbm.at[jj_v.at[b]], rows_v.at[b],
                              semG[b]).wait()

    def combine(b):
        def add_body(e, carry2):
            for k in range(D // 16):
                sl = pl.ds(k * 16, 16)
                rows_v[b, e, sl] = rows_v[b, e, sl] + q_v[b, e, sl]
            return carry2

        lax.fori_loop(0, CHUNK, add_body, 0)

    def scatter_start(b):
        pltpu.async_copy(rows_v.at[b], acc_sh.at[ii_v.at[b]], semS[b],
                         add=True)

    def scatter_wait(b):
        pltpu.make_async_copy(rows_v.at[b], acc_sh.at[ii_v.at[b]],
                              semS[b]).wait()

    for b in (0, 1):
        start_jq(b, b)
        start_i(b, b)

    def body(it, carry):
        c = (2 * it, 2 * it + 1)
        for b in (0, 1):
            wait("j", c[b], b)
            gather_start(b)
        for b in (0, 1):
            gather_wait(b)
            wait("q", c[b], b)
            combine(b)
            wait("i", c[b], b)
            scatter_start(b)
            @pl.when(c[b] + 2 < NCHUNKS)
            def _():
                start_jq(c[b] + 2, b)
        for b in (0, 1):
            scatter_wait(b)

            @pl.when(c[b] + 2 < NCHUNKS)
            def _():
                start_i(c[b] + 2, b)
        return carry

    lax.fori_loop(0, NSTEPS, body, 0)
    cl = NCHUNKS - 1
    wait("j", cl, 0)
    gather_start(0)
    gather_wait(0)
    wait("q", cl, 0)
    combine(0)
    wait("i", cl, 0)
    scatter_start(0)
    scatter_wait(0)
    plsc.subcore_barrier()
    pltpu.sync_copy(
        acc_sh.at[pl.ds(sid * ROWS_PER_TILE, ROWS_PER_TILE)],
        out_hbm.at[pl.ds(cid * NPAD + sid * ROWS_PER_TILE, ROWS_PER_TILE)])


def _sc_scatter(table, ii, jj, q, zeros):
    mesh = plsc.VectorSubcoreMesh(core_axis_name="c", subcore_axis_name="s")
    f = pl.kernel(
        _sc_scatter_body,
        out_type=jax.ShapeDtypeStruct((NC * NPAD, D), jnp.float32),
        mesh=mesh,
        scratch_types=[
            pltpu.VMEM((2, CHUNK), jnp.int32),
            pltpu.VMEM((2, CHUNK), jnp.int32),
            pltpu.VMEM((2, CHUNK, D), jnp.float32),
            pltpu.VMEM((2, CHUNK, D), jnp.float32),
            pltpu.VMEM_SHARED((NPAD, D), jnp.float32),
        ] + [pltpu.SemaphoreType.DMA] * 10,
    )
    return f(table, ii, jj, q, zeros).reshape(NC, NPAD, D)


def kernel(z, edge_index, edge_attr,
           W1_0, b1_0, W2_0, b2_0, We1_0, be1_0, We2_0, be2_0,
           W1_1, b1_1, W2_1, b2_1, We1_1, be1_1, We2_1, be2_1):
    zeros = jnp.zeros((NPAD, D), jnp.float32)
    ii = edge_index[0]
    jj = edge_index[1]

    ea_t = edge_attr.T
    q0 = _edge_mlp(ea_t, We1_0, be1_0, We2_0, be2_0)
    q1 = _edge_mlp(ea_t, We1_1, be1_1, We2_1, be2_1)

    p = _sc_scatter(z, ii, jj, q0, zeros)
    out = _node_update(z, p, W1_0, b1_0, W2_0, b2_0)

    p = _sc_scatter(out, ii, jj, q1, zeros)
    out = _node_update(out, p, W1_1, b1_1, W2_1, b2_1)
    return out
